# Initial kernel scaffold; baseline (speedup 1.0000x reference)
import jax
import jax.numpy as jnp
from jax.experimental import pallas as pl


def kernel(inputs, codebook):
    raise NotImplementedError("write your pallas kernel here")


# speedup vs baseline: 5.2523x; 5.2523x over previous
import jax
import jax.numpy as jnp
from jax.experimental import pallas as pl
from jax.experimental.pallas import tpu as pltpu


def _vq_body(x_ref, cb_ref, loss_ref, q_ref, ppl_ref, oh_ref,
             counts_ref, loss_acc_ref):
    b = pl.program_id(0)
    nb = pl.num_programs(0)
    x = x_ref[0]
    cb = cb_ref[...]

    cx = jax.lax.dot_general(cb, x, (((1,), (0,)), ((), ())),
                             preferred_element_type=jnp.float32)
    x2 = jnp.sum(x * x, axis=0, keepdims=True)
    c2 = jnp.sum(cb * cb, axis=1, keepdims=True)
    dist = (x2 + c2) - 2.0 * cx

    K, N = dist.shape
    idx = jnp.argmin(dist, axis=0)
    kio = jax.lax.broadcasted_iota(jnp.int32, (K, N), 0)
    oh = (kio == idx[None, :]).astype(jnp.float32)
    oh_ref[0] = oh

    q = jax.lax.dot_general(cb, oh, (((0,), (0,)), ((), ())),
                            preferred_element_type=jnp.float32)
    q_ref[0] = q

    dif = q - x
    part = jnp.sum(dif * dif)
    cnt = jnp.sum(oh, axis=1, keepdims=True)

    @pl.when(b == 0)
    def _():
        loss_acc_ref[0, 0] = part
        counts_ref[...] = cnt

    @pl.when(b > 0)
    def _():
        loss_acc_ref[0, 0] = loss_acc_ref[0, 0] + part
        counts_ref[...] = counts_ref[...] + cnt

    @pl.when(b == nb - 1)
    def _():
        d = x_ref.shape[1]
        total_elems = nb * N * d
        loss_ref[0, 0] = loss_acc_ref[0, 0] * (0.25 / total_elems)
        p = counts_ref[...] * (1.0 / (nb * N))
        ent = jnp.sum(p * jnp.log(p + 1e-10))
        ppl_ref[0, 0] = jnp.exp(-ent)


def kernel(inputs, codebook):
    B, D, N = inputs.shape
    K = codebook.shape[0]
    loss2, q, ppl2, oh = pl.pallas_call(
        _vq_body,
        grid=(B,),
        in_specs=[
            pl.BlockSpec((1, D, N), lambda b: (b, 0, 0)),
            pl.BlockSpec((K, D), lambda b: (0, 0)),
        ],
        out_specs=[
            pl.BlockSpec(memory_space=pltpu.SMEM),
            pl.BlockSpec((1, D, N), lambda b: (b, 0, 0)),
            pl.BlockSpec(memory_space=pltpu.SMEM),
            pl.BlockSpec((1, K, N), lambda b: (b, 0, 0)),
        ],
        out_shape=[
            jax.ShapeDtypeStruct((1, 1), jnp.float32),
            jax.ShapeDtypeStruct((B, D, N), jnp.float32),
            jax.ShapeDtypeStruct((1, 1), jnp.float32),
            jax.ShapeDtypeStruct((B, K, N), jnp.float32),
        ],
        scratch_shapes=[
            pltpu.VMEM((K, 1), jnp.float32),
            pltpu.SMEM((1, 1), jnp.float32),
        ],
    )(inputs, codebook)
    return (loss2[0, 0], q, ppl2[0, 0], oh)


# speedup vs baseline: 5.3853x; 1.0253x over previous
import jax
import jax.numpy as jnp
from jax.experimental import pallas as pl
from jax.experimental.pallas import tpu as pltpu


def _vq_body(x_ref, cb_ref, loss_ref, q_ref, ppl_ref, oh_ref,
             counts_ref, loss_acc_ref):
    b = pl.program_id(0)
    nb = pl.num_programs(0)
    x = x_ref[0]
    cb = cb_ref[...]

    cxm = jax.lax.dot_general(cb * -2.0, x, (((1,), (0,)), ((), ())),
                              preferred_element_type=jnp.float32)
    x2 = jnp.sum(x * x, axis=0, keepdims=True)
    c2 = jnp.sum(cb * cb, axis=1, keepdims=True)
    dist = (x2 + c2) + cxm

    K, N = dist.shape
    idx = jnp.argmin(dist, axis=0)
    kio = jax.lax.broadcasted_iota(jnp.int32, (K, N), 0)
    oh = (kio == idx[None, :]).astype(jnp.float32)
    oh_ref[0] = oh

    q = jax.lax.dot_general(cb, oh, (((0,), (0,)), ((), ())),
                            preferred_element_type=jnp.float32)
    q_ref[0] = q

    dif = q - x
    part = jnp.sum(dif * dif)
    cnt = jnp.sum(oh, axis=1, keepdims=True)

    @pl.when(b == 0)
    def _():
        loss_acc_ref[0, 0] = part
        counts_ref[...] = cnt

    @pl.when(b > 0)
    def _():
        loss_acc_ref[0, 0] = loss_acc_ref[0, 0] + part
        counts_ref[...] = counts_ref[...] + cnt

    @pl.when(b == nb - 1)
    def _():
        d = x_ref.shape[1]
        total_elems = nb * N * d
        loss_ref[0, 0] = loss_acc_ref[0, 0] * (0.25 / total_elems)
        p = counts_ref[...] * (1.0 / (nb * N))
        ent = jnp.sum(p * jnp.log(p + 1e-10))
        ppl_ref[0, 0] = jnp.exp(-ent)


def kernel(inputs, codebook):
    B, D, N = inputs.shape
    K = codebook.shape[0]
    loss2, q, ppl2, oh = pl.pallas_call(
        _vq_body,
        grid=(B,),
        in_specs=[
            pl.BlockSpec((1, D, N), lambda b: (b, 0, 0)),
            pl.BlockSpec((K, D), lambda b: (0, 0)),
        ],
        out_specs=[
            pl.BlockSpec(memory_space=pltpu.SMEM),
            pl.BlockSpec((1, D, N), lambda b: (b, 0, 0)),
            pl.BlockSpec(memory_space=pltpu.SMEM),
            pl.BlockSpec((1, K, N), lambda b: (b, 0, 0)),
        ],
        out_shape=[
            jax.ShapeDtypeStruct((1, 1), jnp.float32),
            jax.ShapeDtypeStruct((B, D, N), jnp.float32),
            jax.ShapeDtypeStruct((1, 1), jnp.float32),
            jax.ShapeDtypeStruct((B, K, N), jnp.float32),
        ],
        scratch_shapes=[
            pltpu.VMEM((K, 1), jnp.float32),
            pltpu.SMEM((1, 1), jnp.float32),
        ],
    )(inputs, codebook)
    return (loss2[0, 0], q, ppl2[0, 0], oh)


# speedup vs baseline: 5.9883x; 1.1120x over previous
import jax
import jax.numpy as jnp
from jax.experimental import pallas as pl
from jax.experimental.pallas import tpu as pltpu

_BPS = 2


def _vq_body(x_ref, cb_ref, loss_ref, q_ref, ppl_ref, oh_ref,
             counts_ref, loss_acc_ref):
    b = pl.program_id(0)
    nb = pl.num_programs(0)
    cb = cb_ref[...]
    c2 = jnp.sum(cb * cb, axis=1, keepdims=True)
    cbm2 = cb * -2.0

    part = 0.0
    cnt = None
    for j in range(_BPS):
        x = x_ref[j]
        cxm = jax.lax.dot_general(cbm2, x, (((1,), (0,)), ((), ())),
                                  preferred_element_type=jnp.float32)
        x2 = jnp.sum(x * x, axis=0, keepdims=True)
        dist = (x2 + c2) + cxm

        K, N = dist.shape
        idx = jnp.argmin(dist, axis=0)
        kio = jax.lax.broadcasted_iota(jnp.int32, (K, N), 0)
        oh = (kio == idx[None, :]).astype(jnp.float32)
        oh_ref[j] = oh

        q = jax.lax.dot_general(cb, oh, (((0,), (0,)), ((), ())),
                                preferred_element_type=jnp.float32)
        q_ref[j] = q

        dif = q - x
        part = part + jnp.sum(dif * dif)
        c = jnp.sum(oh, axis=1, keepdims=True)
        cnt = c if cnt is None else cnt + c

    @pl.when(b == 0)
    def _():
        loss_acc_ref[0, 0] = part
        counts_ref[...] = cnt

    @pl.when(b > 0)
    def _():
        loss_acc_ref[0, 0] = loss_acc_ref[0, 0] + part
        counts_ref[...] = counts_ref[...] + cnt

    @pl.when(b == nb - 1)
    def _():
        d = x_ref.shape[1]
        n = x_ref.shape[2]
        total_tok = nb * _BPS * n
        loss_ref[0, 0] = loss_acc_ref[0, 0] * (0.25 / (total_tok * d))
        p = counts_ref[...] * (1.0 / total_tok)
        ent = jnp.sum(p * jnp.log(p + 1e-10))
        ppl_ref[0, 0] = jnp.exp(-ent)


def kernel(inputs, codebook):
    B, D, N = inputs.shape
    K = codebook.shape[0]
    loss2, q, ppl2, oh = pl.pallas_call(
        _vq_body,
        grid=(B // _BPS,),
        in_specs=[
            pl.BlockSpec((_BPS, D, N), lambda b: (b, 0, 0)),
            pl.BlockSpec((K, D), lambda b: (0, 0)),
        ],
        out_specs=[
            pl.BlockSpec(memory_space=pltpu.SMEM),
            pl.BlockSpec((_BPS, D, N), lambda b: (b, 0, 0)),
            pl.BlockSpec(memory_space=pltpu.SMEM),
            pl.BlockSpec((_BPS, K, N), lambda b: (b, 0, 0)),
        ],
        out_shape=[
            jax.ShapeDtypeStruct((1, 1), jnp.float32),
            jax.ShapeDtypeStruct((B, D, N), jnp.float32),
            jax.ShapeDtypeStruct((1, 1), jnp.float32),
            jax.ShapeDtypeStruct((B, K, N), jnp.float32),
        ],
        scratch_shapes=[
            pltpu.VMEM((K, 1), jnp.float32),
            pltpu.SMEM((1, 1), jnp.float32),
        ],
    )(inputs, codebook)
    return (loss2[0, 0], q, ppl2[0, 0], oh)


# speedup vs baseline: 6.1122x; 1.0207x over previous
import jax
import jax.numpy as jnp
from jax.experimental import pallas as pl
from jax.experimental.pallas import tpu as pltpu

_BPS = 4


def _vq_body(x_ref, cb_ref, loss_ref, q_ref, ppl_ref, oh_ref,
             counts_ref, loss_acc_ref):
    b = pl.program_id(0)
    nb = pl.num_programs(0)
    cb = cb_ref[...]
    c2 = jnp.sum(cb * cb, axis=1, keepdims=True)
    cbm2 = cb * -2.0

    part = 0.0
    cnt = None
    for j in range(_BPS):
        x = x_ref[j]
        cxm = jax.lax.dot_general(cbm2, x, (((1,), (0,)), ((), ())),
                                  preferred_element_type=jnp.float32)
        x2 = jnp.sum(x * x, axis=0, keepdims=True)
        dist = (x2 + c2) + cxm

        K, N = dist.shape
        idx = jnp.argmin(dist, axis=0)
        kio = jax.lax.broadcasted_iota(jnp.int32, (K, N), 0)
        oh = (kio == idx[None, :]).astype(jnp.float32)
        oh_ref[j] = oh

        q = jax.lax.dot_general(cb, oh, (((0,), (0,)), ((), ())),
                                preferred_element_type=jnp.float32)
        q_ref[j] = q

        dif = q - x
        part = part + jnp.sum(dif * dif)
        c = jnp.sum(oh, axis=1, keepdims=True)
        cnt = c if cnt is None else cnt + c

    @pl.when(b == 0)
    def _():
        loss_acc_ref[0, 0] = part
        counts_ref[...] = cnt

    @pl.when(b > 0)
    def _():
        loss_acc_ref[0, 0] = loss_acc_ref[0, 0] + part
        counts_ref[...] = counts_ref[...] + cnt

    @pl.when(b == nb - 1)
    def _():
        d = x_ref.shape[1]
        n = x_ref.shape[2]
        total_tok = nb * _BPS * n
        loss_ref[0, 0] = loss_acc_ref[0, 0] * (0.25 / (total_tok * d))
        p = counts_ref[...] * (1.0 / total_tok)
        ent = jnp.sum(p * jnp.log(p + 1e-10))
        ppl_ref[0, 0] = jnp.exp(-ent)


def kernel(inputs, codebook):
    B, D, N = inputs.shape
    K = codebook.shape[0]
    loss2, q, ppl2, oh = pl.pallas_call(
        _vq_body,
        grid=(B // _BPS,),
        in_specs=[
            pl.BlockSpec((_BPS, D, N), lambda b: (b, 0, 0)),
            pl.BlockSpec((K, D), lambda b: (0, 0)),
        ],
        out_specs=[
            pl.BlockSpec(memory_space=pltpu.SMEM),
            pl.BlockSpec((_BPS, D, N), lambda b: (b, 0, 0)),
            pl.BlockSpec(memory_space=pltpu.SMEM),
            pl.BlockSpec((_BPS, K, N), lambda b: (b, 0, 0)),
        ],
        out_shape=[
            jax.ShapeDtypeStruct((1, 1), jnp.float32),
            jax.ShapeDtypeStruct((B, D, N), jnp.float32),
            jax.ShapeDtypeStruct((1, 1), jnp.float32),
            jax.ShapeDtypeStruct((B, K, N), jnp.float32),
        ],
        scratch_shapes=[
            pltpu.VMEM((K, 1), jnp.float32),
            pltpu.SMEM((1, 1), jnp.float32),
        ],
    )(inputs, codebook)
    return (loss2[0, 0], q, ppl2[0, 0], oh)
